# Initial kernel scaffold; baseline (speedup 1.0000x reference)
import jax
import jax.numpy as jnp
from jax.experimental import pallas as pl


def kernel(x, dim, dtype):
    raise NotImplementedError("write your pallas kernel here")


# speedup vs baseline: 2.4855x; 2.4855x over previous
import jax
import jax.numpy as jnp
from jax.experimental import pallas as pl
from jax.experimental.pallas import tpu as pltpu

SEQ_BLK = 256
FEAT_BLK = 2048


def _block_cumsum(blk):
    s = blk.shape[0]
    shift = 1
    while shift < s:
        pad = jnp.zeros((shift, blk.shape[1]), blk.dtype)
        blk = blk + jnp.concatenate([pad, blk[:-shift]], axis=0)
        shift *= 2
    return blk


def _body(x_ref, o_ref, carry_ref):
    sb = pl.program_id(2)

    @pl.when(sb == 0)
    def _():
        carry_ref[...] = jnp.zeros_like(carry_ref)

    acc = _block_cumsum(x_ref[0]) + carry_ref[...]
    o_ref[0] = acc
    carry_ref[...] = acc[SEQ_BLK - 1:SEQ_BLK, :]


def kernel(x, dim, dtype):
    b, s, f = x.shape
    grid = (b, f // FEAT_BLK, s // SEQ_BLK)
    out = pl.pallas_call(
        _body,
        grid=grid,
        in_specs=[pl.BlockSpec((1, SEQ_BLK, FEAT_BLK),
                               lambda b_, f_, s_: (b_, s_, f_))],
        out_specs=pl.BlockSpec((1, SEQ_BLK, FEAT_BLK),
                               lambda b_, f_, s_: (b_, s_, f_)),
        out_shape=jax.ShapeDtypeStruct(x.shape, x.dtype),
        scratch_shapes=[pltpu.VMEM((1, FEAT_BLK), x.dtype)],
        compiler_params=pltpu.CompilerParams(
            dimension_semantics=("parallel", "parallel", "arbitrary"),
        ),
    )(x)
    return out


# speedup vs baseline: 2.8553x; 1.1488x over previous
import jax
import jax.numpy as jnp
from jax.experimental import pallas as pl
from jax.experimental.pallas import tpu as pltpu

SEQ_BLK = 512
FEAT_BLK = 2048


def _block_cumsum(blk):
    s = blk.shape[0]
    shift = 1
    while shift < s:
        pad = jnp.zeros((shift, blk.shape[1]), blk.dtype)
        blk = blk + jnp.concatenate([pad, blk[:-shift]], axis=0)
        shift *= 2
    return blk


def _body(x_ref, o_ref, carry_ref):
    sb = pl.program_id(2)

    @pl.when(sb == 0)
    def _():
        carry_ref[...] = jnp.zeros_like(carry_ref)

    acc = _block_cumsum(x_ref[0]) + carry_ref[...]
    o_ref[0] = acc
    carry_ref[...] = acc[SEQ_BLK - 1:SEQ_BLK, :]


def kernel(x, dim, dtype):
    b, s, f = x.shape
    grid = (b, f // FEAT_BLK, s // SEQ_BLK)
    out = pl.pallas_call(
        _body,
        grid=grid,
        in_specs=[pl.BlockSpec((1, SEQ_BLK, FEAT_BLK),
                               lambda b_, f_, s_: (b_, s_, f_))],
        out_specs=pl.BlockSpec((1, SEQ_BLK, FEAT_BLK),
                               lambda b_, f_, s_: (b_, s_, f_)),
        out_shape=jax.ShapeDtypeStruct(x.shape, x.dtype),
        scratch_shapes=[pltpu.VMEM((1, FEAT_BLK), x.dtype)],
        compiler_params=pltpu.CompilerParams(
            dimension_semantics=("parallel", "parallel", "arbitrary"),
        ),
    )(x)
    return out


# speedup vs baseline: 3.0160x; 1.0563x over previous
import jax
import jax.numpy as jnp
from jax.experimental import pallas as pl
from jax.experimental.pallas import tpu as pltpu

SEQ_BLK = 1024
FEAT_BLK = 2048


def _block_cumsum(blk):
    s = blk.shape[0]
    shift = 1
    while shift < s:
        pad = jnp.zeros((shift, blk.shape[1]), blk.dtype)
        blk = blk + jnp.concatenate([pad, blk[:-shift]], axis=0)
        shift *= 2
    return blk


def _body(x_ref, o_ref, carry_ref):
    sb = pl.program_id(2)

    @pl.when(sb == 0)
    def _():
        carry_ref[...] = jnp.zeros_like(carry_ref)

    acc = _block_cumsum(x_ref[0]) + carry_ref[...]
    o_ref[0] = acc
    carry_ref[...] = acc[SEQ_BLK - 1:SEQ_BLK, :]


def kernel(x, dim, dtype):
    b, s, f = x.shape
    grid = (b, f // FEAT_BLK, s // SEQ_BLK)
    out = pl.pallas_call(
        _body,
        grid=grid,
        in_specs=[pl.BlockSpec((1, SEQ_BLK, FEAT_BLK),
                               lambda b_, f_, s_: (b_, s_, f_))],
        out_specs=pl.BlockSpec((1, SEQ_BLK, FEAT_BLK),
                               lambda b_, f_, s_: (b_, s_, f_)),
        out_shape=jax.ShapeDtypeStruct(x.shape, x.dtype),
        scratch_shapes=[pltpu.VMEM((1, FEAT_BLK), x.dtype)],
        compiler_params=pltpu.CompilerParams(
            dimension_semantics=("parallel", "parallel", "arbitrary"),
        ),
    )(x)
    return out


# speedup vs baseline: 3.5390x; 1.1734x over previous
import jax
import jax.numpy as jnp
from jax.experimental import pallas as pl
from jax.experimental.pallas import tpu as pltpu

SEQ_BLK = 1024
FEAT_BLK = 2048
GROUP = 8


def _group_scan(v):
    s = v.shape[0]
    shift = 1
    while shift < s:
        pad = jnp.zeros((shift, v.shape[1]), v.dtype)
        v = v + jnp.concatenate([pad, v[:-shift]], axis=0)
        shift *= 2
    return v


def _body(x_ref, o_ref, carry_ref):
    sb = pl.program_id(2)

    @pl.when(sb == 0)
    def _():
        carry_ref[...] = jnp.zeros_like(carry_ref)

    def step(g, carry):
        v = x_ref[0, pl.ds(g * GROUP, GROUP), :]
        v = _group_scan(v) + carry
        o_ref[0, pl.ds(g * GROUP, GROUP), :] = v
        return v[GROUP - 1:GROUP, :]

    carry = jax.lax.fori_loop(0, SEQ_BLK // GROUP, step, carry_ref[...],
                              unroll=4)
    carry_ref[...] = carry


def kernel(x, dim, dtype):
    b, s, f = x.shape
    grid = (b, f // FEAT_BLK, s // SEQ_BLK)
    out = pl.pallas_call(
        _body,
        grid=grid,
        in_specs=[pl.BlockSpec((1, SEQ_BLK, FEAT_BLK),
                               lambda b_, f_, s_: (b_, s_, f_))],
        out_specs=pl.BlockSpec((1, SEQ_BLK, FEAT_BLK),
                               lambda b_, f_, s_: (b_, s_, f_)),
        out_shape=jax.ShapeDtypeStruct(x.shape, x.dtype),
        scratch_shapes=[pltpu.VMEM((1, FEAT_BLK), x.dtype)],
        compiler_params=pltpu.CompilerParams(
            dimension_semantics=("parallel", "parallel", "arbitrary"),
        ),
    )(x)
    return out
